# baseline (speedup 1.0000x reference)
import functools

import jax
import jax.numpy as jnp
from jax import lax
from jax.experimental import pallas as pl
from jax.experimental.pallas import tpu as pltpu
from jax.experimental.pallas import tpu_sc as plsc

_KH = 32
_KW = 32
_THRESH = 0.5
_TILE_N = 400


def _tile_topone(tile, q_bf, base, n_db, dim):
    tile_bf = tile.astype(jnp.bfloat16)
    dots = lax.dot_general(tile_bf, q_bf, (((1,), (1,)), ((), ())),
                           preferred_element_type=jnp.float32)
    ones_rhs = jnp.ones((8, dim), jnp.bfloat16)
    norms = lax.dot_general(tile_bf * tile_bf, ones_rhs,
                            (((1,), (1,)), ((), ())),
                            preferred_element_type=jnp.float32)
    score = norms[:, :1] - 2.0 * dots
    row = base + lax.broadcasted_iota(jnp.int32, score.shape, 0)
    score = jnp.where(row < n_db, score, jnp.inf)
    local_min = jnp.min(score, axis=0, keepdims=True)
    cand = jnp.where(score == local_min, row, n_db)
    local_arg = jnp.min(cand, axis=0, keepdims=True)
    return local_min, local_arg


def _search_body(patches_ref, mem_a_ref, mem_b_ref, dist_ref, idx_ref, *,
                 n_db, n_half, tile_n, dim):
    g = pl.program_id(0)
    q = patches_ref[...]
    q_bf = q.astype(jnp.bfloat16)
    min_a, arg_a = _tile_topone(mem_a_ref[...], q_bf, g * tile_n, n_db, dim)
    min_b, arg_b = _tile_topone(mem_b_ref[...], q_bf, n_half + g * tile_n,
                                n_db, dim)
    b_wins = min_b < min_a
    local_min = jnp.where(b_wins, min_b, min_a)
    local_arg = jnp.where(b_wins, arg_b, arg_a)

    @pl.when(g == 0)
    def _():
        dist_ref[...] = local_min
        idx_ref[...] = local_arg

    @pl.when(g > 0)
    def _():
        prev = dist_ref[...]
        better = local_min < prev
        dist_ref[...] = jnp.where(better, local_min, prev)
        idx_ref[...] = jnp.where(better, local_arg, idx_ref[...])

    @pl.when(g == pl.num_programs(0) - 1)
    def _():
        qsq = jnp.sum(q * q, axis=1)[None, :]
        dist_ref[...] = dist_ref[...] + qsq


def _search(patches, mem, tile_n=_TILE_N):
    p, dim = patches.shape
    n_db = mem.shape[0]
    n_half = (n_db + 1) // 2
    grid = (n_half + tile_n - 1) // tile_n
    half_blocks = n_half // tile_n
    dist, idx = pl.pallas_call(
        functools.partial(_search_body, n_db=n_db, n_half=n_half,
                          tile_n=tile_n, dim=dim),
        grid=(grid,),
        in_specs=[
            pl.BlockSpec((p, dim), lambda g: (0, 0)),
            pl.BlockSpec((tile_n, dim), lambda g: (g, 0)),
            pl.BlockSpec((tile_n, dim),
                         lambda g: (half_blocks + g, 0)),
        ],
        out_specs=[
            pl.BlockSpec((1, p), lambda g: (0, 0)),
            pl.BlockSpec((1, p), lambda g: (0, 0)),
        ],
        out_shape=[
            jax.ShapeDtypeStruct((1, p), jnp.float32),
            jax.ShapeDtypeStruct((1, p), jnp.int32),
        ],
    )(patches, mem, mem)
    return dist[0], idx[0]


def _gather_recon(idx, mapping, mem2):
    n_rows = idx.shape[0]
    dim = mem2.shape[1]
    info = plsc.get_sparse_core_info()
    nc = info.num_cores
    rows_per_worker = 8
    n_workers = n_rows // rows_per_worker
    mesh = plsc.VectorSubcoreMesh(core_axis_name="c", subcore_axis_name="s")

    @functools.partial(
        pl.kernel,
        mesh=mesh,
        out_type=jax.ShapeDtypeStruct((n_rows, dim), jnp.float32),
        scratch_types=[
            pltpu.VMEM((rows_per_worker,), jnp.int32),
            pltpu.VMEM((rows_per_worker,), jnp.int32),
            pltpu.VMEM((rows_per_worker, dim), jnp.float32),
            pltpu.SemaphoreType.DMA,
        ],
    )
    def k(idx_hbm, mapping_hbm, mem2_hbm, out_hbm, idx_v, idx2_v, rows_v, sem):
        wid = lax.axis_index("s") * nc + lax.axis_index("c")

        @pl.when(wid < n_workers)
        def _():
            base = wid * rows_per_worker
            pltpu.sync_copy(idx_hbm.at[pl.ds(base, rows_per_worker)], idx_v)
            pltpu.async_copy(mapping_hbm.at[idx_v], idx2_v, sem).wait()
            pltpu.async_copy(mem2_hbm.at[idx2_v], rows_v, sem).wait()
            pltpu.sync_copy(rows_v, out_hbm.at[pl.ds(base, rows_per_worker)])

    return k(idx, mapping, mem2)


def kernel(image, mem, mem2, mapping):
    h, w, c = image.shape
    oh, ow = h // _KH, w // _KW
    dim = c * _KH * _KW
    img = jnp.transpose(image, (2, 0, 1))
    patches = (img.reshape(c, oh, _KH, ow, _KW)
               .transpose(1, 3, 0, 2, 4)
               .reshape(oh * ow, dim))
    dist, idx = _search(patches, mem)
    recon = _gather_recon(idx, mapping, mem2)
    out_patches = jnp.where((dist >= _THRESH)[:, None], patches, recon)
    folded = (out_patches.reshape(oh, ow, c, _KH, _KW)
              .transpose(2, 0, 3, 1, 4)
              .reshape(c, h, w))
    folded = folded / jnp.max(folded)
    return jnp.transpose(folded, (1, 2, 0))


# speedup vs baseline: 1.1184x; 1.1184x over previous
import functools

import jax
import jax.numpy as jnp
from jax import lax
from jax.experimental import pallas as pl
from jax.experimental.pallas import tpu as pltpu
from jax.experimental.pallas import tpu_sc as plsc

_KH = 32
_KW = 32
_THRESH = 0.5
_TILE_N = 800


def _tile_topone(tile, q_bf, base, n_db, dim):
    tile_bf = tile.astype(jnp.bfloat16)
    dots = lax.dot_general(tile_bf, q_bf, (((1,), (1,)), ((), ())),
                           preferred_element_type=jnp.float32)
    norms = jnp.sum(tile * tile, axis=1)
    score = norms[:, None] - 2.0 * dots
    row = base + lax.broadcasted_iota(jnp.int32, score.shape, 0)
    score = jnp.where(row < n_db, score, jnp.inf)
    local_min = jnp.min(score, axis=0, keepdims=True)
    cand = jnp.where(score == local_min, row, n_db)
    local_arg = jnp.min(cand, axis=0, keepdims=True)
    return local_min, local_arg


def _search_body(patches_ref, mem_a_ref, mem_b_ref, dist_ref, idx_ref, *,
                 n_db, n_half, tile_n, dim):
    g = pl.program_id(0)
    q = patches_ref[...]
    q_bf = q.astype(jnp.bfloat16)
    min_a, arg_a = _tile_topone(mem_a_ref[...], q_bf, g * tile_n, n_db, dim)
    min_b, arg_b = _tile_topone(mem_b_ref[...], q_bf, n_half + g * tile_n,
                                n_db, dim)
    b_wins = min_b < min_a
    local_min = jnp.where(b_wins, min_b, min_a)
    local_arg = jnp.where(b_wins, arg_b, arg_a)

    @pl.when(g == 0)
    def _():
        dist_ref[...] = local_min
        idx_ref[...] = local_arg

    @pl.when(g > 0)
    def _():
        prev = dist_ref[...]
        better = local_min < prev
        dist_ref[...] = jnp.where(better, local_min, prev)
        idx_ref[...] = jnp.where(better, local_arg, idx_ref[...])

    @pl.when(g == pl.num_programs(0) - 1)
    def _():
        qsq = jnp.sum(q * q, axis=1)[None, :]
        dist_ref[...] = dist_ref[...] + qsq


def _search(patches, mem, tile_n=_TILE_N):
    p, dim = patches.shape
    n_db = mem.shape[0]
    total_blocks = (n_db + tile_n - 1) // tile_n
    half_blocks = total_blocks // 2
    n_half = half_blocks * tile_n
    grid = total_blocks - half_blocks
    dist, idx = pl.pallas_call(
        functools.partial(_search_body, n_db=n_db, n_half=n_half,
                          tile_n=tile_n, dim=dim),
        grid=(grid,),
        in_specs=[
            pl.BlockSpec((p, dim), lambda g: (0, 0)),
            pl.BlockSpec((tile_n, dim), lambda g: (g, 0)),
            pl.BlockSpec((tile_n, dim),
                         lambda g: (half_blocks + g, 0)),
        ],
        out_specs=[
            pl.BlockSpec((1, p), lambda g: (0, 0)),
            pl.BlockSpec((1, p), lambda g: (0, 0)),
        ],
        out_shape=[
            jax.ShapeDtypeStruct((1, p), jnp.float32),
            jax.ShapeDtypeStruct((1, p), jnp.int32),
        ],
    )(patches, mem, mem)
    return dist[0], idx[0]


def _gather_recon(idx, mapping, mem2):
    n_rows = idx.shape[0]
    dim = mem2.shape[1]
    info = plsc.get_sparse_core_info()
    nc = info.num_cores
    rows_per_worker = 8
    n_workers = n_rows // rows_per_worker
    mesh = plsc.VectorSubcoreMesh(core_axis_name="c", subcore_axis_name="s")

    @functools.partial(
        pl.kernel,
        mesh=mesh,
        out_type=jax.ShapeDtypeStruct((n_rows, dim), jnp.float32),
        scratch_types=[
            pltpu.VMEM((rows_per_worker,), jnp.int32),
            pltpu.VMEM((rows_per_worker,), jnp.int32),
            pltpu.VMEM((rows_per_worker, dim), jnp.float32),
            pltpu.SemaphoreType.DMA,
        ],
    )
    def k(idx_hbm, mapping_hbm, mem2_hbm, out_hbm, idx_v, idx2_v, rows_v, sem):
        wid = lax.axis_index("s") * nc + lax.axis_index("c")

        @pl.when(wid < n_workers)
        def _():
            base = wid * rows_per_worker
            pltpu.sync_copy(idx_hbm.at[pl.ds(base, rows_per_worker)], idx_v)
            pltpu.async_copy(mapping_hbm.at[idx_v], idx2_v, sem).wait()
            pltpu.async_copy(mem2_hbm.at[idx2_v], rows_v, sem).wait()
            pltpu.sync_copy(rows_v, out_hbm.at[pl.ds(base, rows_per_worker)])

    return k(idx, mapping, mem2)


def kernel(image, mem, mem2, mapping):
    h, w, c = image.shape
    oh, ow = h // _KH, w // _KW
    dim = c * _KH * _KW
    img = jnp.transpose(image, (2, 0, 1))
    patches = (img.reshape(c, oh, _KH, ow, _KW)
               .transpose(1, 3, 0, 2, 4)
               .reshape(oh * ow, dim))
    dist, idx = _search(patches, mem)
    recon = _gather_recon(idx, mapping, mem2)
    out_patches = jnp.where((dist >= _THRESH)[:, None], patches, recon)
    folded = (out_patches.reshape(oh, ow, c, _KH, _KW)
              .transpose(2, 0, 3, 1, 4)
              .reshape(c, h, w))
    folded = folded / jnp.max(folded)
    return jnp.transpose(folded, (1, 2, 0))


# speedup vs baseline: 1.1392x; 1.0187x over previous
import functools

import jax
import jax.numpy as jnp
from jax import lax
from jax.experimental import pallas as pl
from jax.experimental.pallas import tpu as pltpu
from jax.experimental.pallas import tpu_sc as plsc

_KH = 32
_KW = 32
_THRESH = 0.5
_TILE_N = 400


def _tile_topone(tile, q_bf, base, n_db, dim):
    tile_bf = tile.astype(jnp.bfloat16)
    dots = lax.dot_general(tile_bf, q_bf, (((1,), (1,)), ((), ())),
                           preferred_element_type=jnp.float32)
    norms = jnp.sum(tile * tile, axis=1)
    score = norms[:, None] - 2.0 * dots
    row = base + lax.broadcasted_iota(jnp.int32, score.shape, 0)
    score = jnp.where(row < n_db, score, jnp.inf)
    local_min = jnp.min(score, axis=0, keepdims=True)
    cand = jnp.where(score == local_min, row, n_db)
    local_arg = jnp.min(cand, axis=0, keepdims=True)
    return local_min, local_arg


def _search_body(patches_ref, *refs, n_db, stream_starts, tile_n, dim):
    n_streams = len(stream_starts)
    mem_refs = refs[:n_streams]
    dist_ref, idx_ref = refs[n_streams:]
    g = pl.program_id(0)
    q = patches_ref[...]
    q_bf = q.astype(jnp.bfloat16)
    local_min, local_arg = None, None
    for ref, start in zip(mem_refs, stream_starts):
        lmin, larg = _tile_topone(ref[...], q_bf, start + g * tile_n,
                                  n_db, dim)
        if local_min is None:
            local_min, local_arg = lmin, larg
        else:
            wins = lmin < local_min
            local_min = jnp.where(wins, lmin, local_min)
            local_arg = jnp.where(wins, larg, local_arg)

    @pl.when(g == 0)
    def _():
        dist_ref[...] = local_min
        idx_ref[...] = local_arg

    @pl.when(g > 0)
    def _():
        prev = dist_ref[...]
        better = local_min < prev
        dist_ref[...] = jnp.where(better, local_min, prev)
        idx_ref[...] = jnp.where(better, local_arg, idx_ref[...])

    @pl.when(g == pl.num_programs(0) - 1)
    def _():
        qsq = jnp.sum(q * q, axis=1)[None, :]
        dist_ref[...] = dist_ref[...] + qsq


def _search(patches, mem, tile_n=_TILE_N, n_streams=4):
    p, dim = patches.shape
    n_db = mem.shape[0]
    total_blocks = (n_db + tile_n - 1) // tile_n
    start_blocks = [i * total_blocks // n_streams for i in range(n_streams)]
    grid = total_blocks - start_blocks[-1]
    stream_starts = [b * tile_n for b in start_blocks]

    def make_index_map(start_block):
        return lambda g, s=start_block: (s + g, 0)

    dist, idx = pl.pallas_call(
        functools.partial(_search_body, n_db=n_db,
                          stream_starts=stream_starts,
                          tile_n=tile_n, dim=dim),
        grid=(grid,),
        in_specs=[pl.BlockSpec((p, dim), lambda g: (0, 0))] + [
            pl.BlockSpec((tile_n, dim), make_index_map(s))
            for s in start_blocks
        ],
        out_specs=[
            pl.BlockSpec((1, p), lambda g: (0, 0)),
            pl.BlockSpec((1, p), lambda g: (0, 0)),
        ],
        out_shape=[
            jax.ShapeDtypeStruct((1, p), jnp.float32),
            jax.ShapeDtypeStruct((1, p), jnp.int32),
        ],
    )(patches, *([mem] * n_streams))
    return dist[0], idx[0]


def _gather_recon(idx, mapping, mem2):
    n_rows = idx.shape[0]
    dim = mem2.shape[1]
    info = plsc.get_sparse_core_info()
    nc = info.num_cores
    rows_per_worker = 8
    n_workers = n_rows // rows_per_worker
    mesh = plsc.VectorSubcoreMesh(core_axis_name="c", subcore_axis_name="s")

    @functools.partial(
        pl.kernel,
        mesh=mesh,
        out_type=jax.ShapeDtypeStruct((n_rows, dim), jnp.float32),
        scratch_types=[
            pltpu.VMEM((rows_per_worker,), jnp.int32),
            pltpu.VMEM((rows_per_worker,), jnp.int32),
            pltpu.VMEM((rows_per_worker, dim), jnp.float32),
            pltpu.SemaphoreType.DMA,
        ],
    )
    def k(idx_hbm, mapping_hbm, mem2_hbm, out_hbm, idx_v, idx2_v, rows_v, sem):
        wid = lax.axis_index("s") * nc + lax.axis_index("c")

        @pl.when(wid < n_workers)
        def _():
            base = wid * rows_per_worker
            pltpu.sync_copy(idx_hbm.at[pl.ds(base, rows_per_worker)], idx_v)
            pltpu.async_copy(mapping_hbm.at[idx_v], idx2_v, sem).wait()
            pltpu.async_copy(mem2_hbm.at[idx2_v], rows_v, sem).wait()
            pltpu.sync_copy(rows_v, out_hbm.at[pl.ds(base, rows_per_worker)])

    return k(idx, mapping, mem2)


def kernel(image, mem, mem2, mapping):
    h, w, c = image.shape
    oh, ow = h // _KH, w // _KW
    dim = c * _KH * _KW
    img = jnp.transpose(image, (2, 0, 1))
    patches = (img.reshape(c, oh, _KH, ow, _KW)
               .transpose(1, 3, 0, 2, 4)
               .reshape(oh * ow, dim))
    dist, idx = _search(patches, mem)
    recon = _gather_recon(idx, mapping, mem2)
    out_patches = jnp.where((dist >= _THRESH)[:, None], patches, recon)
    folded = (out_patches.reshape(oh, ow, c, _KH, _KW)
              .transpose(2, 0, 3, 1, 4)
              .reshape(c, h, w))
    folded = folded / jnp.max(folded)
    return jnp.transpose(folded, (1, 2, 0))


# speedup vs baseline: 1.1432x; 1.0034x over previous
import functools

import jax
import jax.numpy as jnp
from jax import lax
from jax.experimental import pallas as pl
from jax.experimental.pallas import tpu as pltpu
from jax.experimental.pallas import tpu_sc as plsc

_KH = 32
_KW = 32
_THRESH = 0.5
_TILE_N = 200


def _tile_topone(tile, q_bf, base, n_db, dim):
    tile_bf = tile.astype(jnp.bfloat16)
    dots = lax.dot_general(tile_bf, q_bf, (((1,), (1,)), ((), ())),
                           preferred_element_type=jnp.float32)
    norms = jnp.sum(tile * tile, axis=1)
    score = norms[:, None] - 2.0 * dots
    row = base + lax.broadcasted_iota(jnp.int32, score.shape, 0)
    score = jnp.where(row < n_db, score, jnp.inf)
    local_min = jnp.min(score, axis=0, keepdims=True)
    cand = jnp.where(score == local_min, row, n_db)
    local_arg = jnp.min(cand, axis=0, keepdims=True)
    return local_min, local_arg


def _search_body(patches_ref, *refs, n_db, stream_starts, tile_n, dim):
    n_streams = len(stream_starts)
    mem_refs = refs[:n_streams]
    dist_ref, idx_ref = refs[n_streams:]
    g = pl.program_id(0)
    q = patches_ref[...]
    q_bf = q.astype(jnp.bfloat16)
    local_min, local_arg = None, None
    for ref, start in zip(mem_refs, stream_starts):
        lmin, larg = _tile_topone(ref[...], q_bf, start + g * tile_n,
                                  n_db, dim)
        if local_min is None:
            local_min, local_arg = lmin, larg
        else:
            wins = lmin < local_min
            local_min = jnp.where(wins, lmin, local_min)
            local_arg = jnp.where(wins, larg, local_arg)

    @pl.when(g == 0)
    def _():
        dist_ref[...] = local_min
        idx_ref[...] = local_arg

    @pl.when(g > 0)
    def _():
        prev = dist_ref[...]
        better = local_min < prev
        dist_ref[...] = jnp.where(better, local_min, prev)
        idx_ref[...] = jnp.where(better, local_arg, idx_ref[...])

    @pl.when(g == pl.num_programs(0) - 1)
    def _():
        qsq = jnp.sum(q * q, axis=1)[None, :]
        dist_ref[...] = dist_ref[...] + qsq


def _search(patches, mem, tile_n=_TILE_N, n_streams=8):
    p, dim = patches.shape
    n_db = mem.shape[0]
    total_blocks = (n_db + tile_n - 1) // tile_n
    start_blocks = [i * total_blocks // n_streams for i in range(n_streams)]
    grid = total_blocks - start_blocks[-1]
    stream_starts = [b * tile_n for b in start_blocks]

    def make_index_map(start_block):
        return lambda g, s=start_block: (s + g, 0)

    dist, idx = pl.pallas_call(
        functools.partial(_search_body, n_db=n_db,
                          stream_starts=stream_starts,
                          tile_n=tile_n, dim=dim),
        grid=(grid,),
        in_specs=[pl.BlockSpec((p, dim), lambda g: (0, 0))] + [
            pl.BlockSpec((tile_n, dim), make_index_map(s))
            for s in start_blocks
        ],
        out_specs=[
            pl.BlockSpec((1, p), lambda g: (0, 0)),
            pl.BlockSpec((1, p), lambda g: (0, 0)),
        ],
        out_shape=[
            jax.ShapeDtypeStruct((1, p), jnp.float32),
            jax.ShapeDtypeStruct((1, p), jnp.int32),
        ],
    )(patches, *([mem] * n_streams))
    return dist[0], idx[0]


def _gather_recon(idx, mapping, mem2):
    n_rows = idx.shape[0]
    dim = mem2.shape[1]
    info = plsc.get_sparse_core_info()
    nc = info.num_cores
    rows_per_worker = 8
    n_workers = n_rows // rows_per_worker
    mesh = plsc.VectorSubcoreMesh(core_axis_name="c", subcore_axis_name="s")

    @functools.partial(
        pl.kernel,
        mesh=mesh,
        out_type=jax.ShapeDtypeStruct((n_rows, dim), jnp.float32),
        scratch_types=[
            pltpu.VMEM((rows_per_worker,), jnp.int32),
            pltpu.VMEM((rows_per_worker,), jnp.int32),
            pltpu.VMEM((rows_per_worker, dim), jnp.float32),
            pltpu.SemaphoreType.DMA,
        ],
    )
    def k(idx_hbm, mapping_hbm, mem2_hbm, out_hbm, idx_v, idx2_v, rows_v, sem):
        wid = lax.axis_index("s") * nc + lax.axis_index("c")

        @pl.when(wid < n_workers)
        def _():
            base = wid * rows_per_worker
            pltpu.sync_copy(idx_hbm.at[pl.ds(base, rows_per_worker)], idx_v)
            pltpu.async_copy(mapping_hbm.at[idx_v], idx2_v, sem).wait()
            pltpu.async_copy(mem2_hbm.at[idx2_v], rows_v, sem).wait()
            pltpu.sync_copy(rows_v, out_hbm.at[pl.ds(base, rows_per_worker)])

    return k(idx, mapping, mem2)


def kernel(image, mem, mem2, mapping):
    h, w, c = image.shape
    oh, ow = h // _KH, w // _KW
    dim = c * _KH * _KW
    img = jnp.transpose(image, (2, 0, 1))
    patches = (img.reshape(c, oh, _KH, ow, _KW)
               .transpose(1, 3, 0, 2, 4)
               .reshape(oh * ow, dim))
    dist, idx = _search(patches, mem)
    recon = _gather_recon(idx, mapping, mem2)
    out_patches = jnp.where((dist >= _THRESH)[:, None], patches, recon)
    folded = (out_patches.reshape(oh, ow, c, _KH, _KW)
              .transpose(2, 0, 3, 1, 4)
              .reshape(c, h, w))
    folded = folded / jnp.max(folded)
    return jnp.transpose(folded, (1, 2, 0))
